# Initial kernel scaffold; baseline (speedup 1.0000x reference)
import jax
import jax.numpy as jnp
from jax.experimental import pallas as pl


def kernel(inputs, emb_table, W1, b1, W2, b2):
    raise NotImplementedError("write your pallas kernel here")


# speedup vs baseline: 1.0049x; 1.0049x over previous
import functools

import jax
import jax.numpy as jnp
from jax import lax
from jax.experimental import pallas as pl
from jax.experimental.pallas import tpu as pltpu
from jax.experimental.pallas import tpu_sc as plsc

VOCAB = 100000
EMBED_DIM = 64
CONTEXT = 20
BATCH = 4096
HIDDEN = 128

NUM_WORKERS = 32
TOTAL_LOOKUPS = BATCH * CONTEXT
LOOKUPS_PER_WORKER = TOTAL_LOOKUPS // NUM_WORKERS
IDX_CHUNK = 128
CHUNKS_PER_WORKER = LOOKUPS_PER_WORKER // IDX_CHUNK
HALF = CHUNKS_PER_WORKER // 2

TILE_B = 32


def _sc_gather_kernel(idx_hbm, table_hbm, out_hbm, idx_v, rows_v, sem):
    wid = lax.axis_index("s") * 2 + lax.axis_index("c")
    pltpu.sync_copy(idx_hbm.at[wid], idx_v)
    for half in range(2):
        cps = []
        for j in range(HALF):
            chunk = half * HALF + j
            cps.append(
                pltpu.async_copy(
                    table_hbm.at[idx_v.at[chunk]],
                    rows_v.at[pl.ds(j * IDX_CHUNK, IDX_CHUNK)],
                    sem,
                )
            )
        for cp in cps:
            cp.wait()
        base = wid * LOOKUPS_PER_WORKER + half * HALF * IDX_CHUNK
        pltpu.sync_copy(rows_v, out_hbm.at[pl.ds(base, HALF * IDX_CHUNK)])


def _sc_gather(idx, table):
    mesh = plsc.VectorSubcoreMesh(core_axis_name="c", subcore_axis_name="s")
    return pl.kernel(
        _sc_gather_kernel,
        mesh=mesh,
        out_type=jax.ShapeDtypeStruct((TOTAL_LOOKUPS, EMBED_DIM), jnp.float32),
        scratch_types=[
            pltpu.VMEM((CHUNKS_PER_WORKER, IDX_CHUNK), jnp.int32),
            pltpu.VMEM((HALF * IDX_CHUNK, EMBED_DIM), jnp.float32),
            pltpu.SemaphoreType.DMA,
        ],
        compiler_params=pltpu.CompilerParams(use_tc_tiling_on_sc=False),
    )(idx, table)


def _tc_fused_kernel(emb_ref, w1_ref, b1_ref, w2_ref, b2_ref, out_ref):
    e = emb_ref[...].astype(jnp.bfloat16)
    h = jnp.dot(e, w1_ref[...], preferred_element_type=jnp.float32)
    h = jnp.maximum(h + b1_ref[...], 0.0).astype(jnp.bfloat16)
    logits = jnp.dot(h, w2_ref[...], preferred_element_type=jnp.float32)
    logits = logits + b2_ref[...]
    m = jnp.max(logits, axis=1, keepdims=True)
    s = jnp.sum(jnp.exp(logits - m), axis=1, keepdims=True)
    out_ref[...] = logits - (m + jnp.log(s))


def _tc_fused(embeds, W1t, b1, W2t, b2):
    grid = (BATCH // TILE_B,)
    return pl.pallas_call(
        _tc_fused_kernel,
        grid=grid,
        in_specs=[
            pl.BlockSpec((TILE_B, CONTEXT * EMBED_DIM), lambda i: (i, 0)),
            pl.BlockSpec((CONTEXT * EMBED_DIM, HIDDEN), lambda i: (0, 0)),
            pl.BlockSpec((1, HIDDEN), lambda i: (0, 0)),
            pl.BlockSpec((HIDDEN, VOCAB), lambda i: (0, 0)),
            pl.BlockSpec((1, VOCAB), lambda i: (0, 0)),
        ],
        out_specs=pl.BlockSpec((TILE_B, VOCAB), lambda i: (i, 0)),
        out_shape=jax.ShapeDtypeStruct((BATCH, VOCAB), jnp.float32),
        compiler_params=pltpu.CompilerParams(
            dimension_semantics=("arbitrary",),
            vmem_limit_bytes=128 * 1024 * 1024,
        ),
    )(embeds, W1t, b1, W2t, b2)


def kernel(inputs, emb_table, W1, b1, W2, b2):
    idx = inputs.reshape(NUM_WORKERS, CHUNKS_PER_WORKER, IDX_CHUNK)
    embeds = _sc_gather(idx, emb_table).reshape(BATCH, CONTEXT * EMBED_DIM)
    W1t = W1.T.astype(jnp.bfloat16)
    W2t = W2.T.astype(jnp.bfloat16)
    return _tc_fused(embeds, W1t, b1.reshape(1, HIDDEN), W2t, b2.reshape(1, VOCAB))
